# Initial kernel scaffold; baseline (speedup 1.0000x reference)
import jax
import jax.numpy as jnp
from jax.experimental import pallas as pl


def kernel(z2_onehot, W, embedding_table):
    raise NotImplementedError("write your pallas kernel here")


# speedup vs baseline: 1.4401x; 1.4401x over previous
import functools

import jax
import jax.numpy as jnp
from jax import lax
from jax.experimental import pallas as pl

_NUM_MODES = 64
_BATCH = 16384
_ROWS = 512

_KS = (0, 42, 0x1BD11BDA ^ 42)
_ROT = ((13, 15, 26, 6), (17, 29, 16, 24))


def _threefry_bits(p):
    ks = tuple(jnp.uint32(k) for k in _KS)
    x0 = jnp.zeros_like(p) + ks[0]
    x1 = p + ks[1]
    for i in range(5):
        for r in _ROT[i % 2]:
            x0 = x0 + x1
            x1 = (x1 << jnp.uint32(r)) | (x1 >> jnp.uint32(32 - r))
            x1 = x0 ^ x1
        x0 = x0 + ks[(i + 1) % 3]
        x1 = x1 + ks[(i + 2) % 3] + jnp.uint32(i + 1)
    return x0 ^ x1


def _sample_body(z_ref, wt_ref, tabt_ref, out_ref):
    shp = (_ROWS, _NUM_MODES)
    base = (pl.program_id(0) * _ROWS).astype(jnp.uint32)
    row = lax.broadcasted_iota(jnp.uint32, shp, 0) + base
    col = lax.broadcasted_iota(jnp.uint32, shp, 1)
    p = row * jnp.uint32(_NUM_MODES) + col

    bits = _threefry_bits(p)
    fb = (bits >> jnp.uint32(9)) | jnp.uint32(0x3F800000)
    u = lax.bitcast_convert_type(fb, jnp.float32) - jnp.float32(1.0)
    tiny = jnp.float32(jnp.finfo(jnp.float32).tiny)
    unif = jnp.maximum(tiny, u * (jnp.float32(1.0) - tiny) + tiny)
    g = -jnp.log(-jnp.log(unif))

    def _b(x):
        return x.astype(jnp.bfloat16).astype(jnp.float32)

    logits = (_b(z_ref[:, 0:1]) * _b(wt_ref[0:1, :])
              + _b(z_ref[:, 1:2]) * _b(wt_ref[1:2, :]))
    val = g + logits

    m = jnp.max(val, axis=1, keepdims=True)
    coli = lax.broadcasted_iota(jnp.int32, shp, 1)
    cand = jnp.where(val == m, coli, jnp.int32(_NUM_MODES))
    idx = jnp.min(cand, axis=1, keepdims=True)

    onehot = ((val == m) & (coli == idx)).astype(jnp.float32)
    e0 = jnp.sum(onehot * tabt_ref[0:1, :], axis=1, keepdims=True)
    e1 = jnp.sum(onehot * tabt_ref[1:2, :], axis=1, keepdims=True)
    out_ref[...] = jnp.concatenate([e0, e1], axis=1)


@functools.partial(jax.jit, static_argnames=("interpret",))
def _run(z2_onehot, W, embedding_table, interpret=False):
    wt = W.T
    tabt = embedding_table.T
    grid = (_BATCH // _ROWS,)
    out = pl.pallas_call(
        _sample_body,
        grid=grid,
        in_specs=[
            pl.BlockSpec((_ROWS, 2), lambda i: (i, 0)),
            pl.BlockSpec((2, _NUM_MODES), lambda i: (0, 0)),
            pl.BlockSpec((2, _NUM_MODES), lambda i: (0, 0)),
        ],
        out_specs=pl.BlockSpec((_ROWS, 2), lambda i: (i, 0)),
        out_shape=jax.ShapeDtypeStruct((_BATCH, 2), jnp.float32),
        interpret=interpret,
    )(z2_onehot, wt, tabt)
    return out


def kernel(z2_onehot, W, embedding_table):
    return _run(z2_onehot, W, embedding_table)


# speedup vs baseline: 2.6693x; 1.8535x over previous
import functools

import jax
import jax.numpy as jnp
from jax import lax
from jax.experimental import pallas as pl

_NUM_MODES = 64
_BATCH = 16384
_COLS = 2048

_KS = (0, 42, 0x1BD11BDA ^ 42)
_ROT = ((13, 15, 26, 6), (17, 29, 16, 24))


def _threefry_bits(p):
    ks = tuple(jnp.uint32(k) for k in _KS)
    x0 = jnp.zeros_like(p) + ks[0]
    x1 = p + ks[1]
    for i in range(5):
        for r in _ROT[i % 2]:
            x0 = x0 + x1
            x1 = (x1 << jnp.uint32(r)) | (x1 >> jnp.uint32(32 - r))
            x1 = x0 ^ x1
        x0 = x0 + ks[(i + 1) % 3]
        x1 = x1 + ks[(i + 2) % 3] + jnp.uint32(i + 1)
    return x0 ^ x1


def _sample_body(zt_ref, w_ref, tab_ref, out_ref):
    shp = (_NUM_MODES, _COLS)
    base = (pl.program_id(0) * _COLS).astype(jnp.uint32)
    lane = lax.broadcasted_iota(jnp.uint32, shp, 1) + base
    mode = lax.broadcasted_iota(jnp.uint32, shp, 0)
    p = lane * jnp.uint32(_NUM_MODES) + mode

    bits = _threefry_bits(p)
    fb = (bits >> jnp.uint32(9)) | jnp.uint32(0x3F800000)
    u = lax.bitcast_convert_type(fb, jnp.float32) - jnp.float32(1.0)
    tiny = jnp.float32(jnp.finfo(jnp.float32).tiny)
    unif = jnp.maximum(tiny, u * (jnp.float32(1.0) - tiny) + tiny)
    g = -jnp.log(-jnp.log(unif))

    def _b(x):
        return x.astype(jnp.bfloat16).astype(jnp.float32)

    logits = (_b(zt_ref[0:1, :]) * _b(w_ref[:, 0:1])
              + _b(zt_ref[1:2, :]) * _b(w_ref[:, 1:2]))
    val = g + logits

    m = jnp.max(val, axis=0, keepdims=True)
    modei = lax.broadcasted_iota(jnp.int32, shp, 0)
    cand = jnp.where(val == m, modei, jnp.int32(_NUM_MODES))
    idx = jnp.min(cand, axis=0, keepdims=True)

    onehot = ((val == m) & (modei == idx)).astype(jnp.float32)
    e0 = jnp.sum(onehot * tab_ref[:, 0:1], axis=0, keepdims=True)
    e1 = jnp.sum(onehot * tab_ref[:, 1:2], axis=0, keepdims=True)
    out_ref[...] = jnp.concatenate([e0, e1], axis=0)


@functools.partial(jax.jit, static_argnames=("interpret",))
def _run(z2_onehot, W, embedding_table, interpret=False):
    zt = z2_onehot.T
    grid = (_BATCH // _COLS,)
    out = pl.pallas_call(
        _sample_body,
        grid=grid,
        in_specs=[
            pl.BlockSpec((2, _COLS), lambda i: (0, i)),
            pl.BlockSpec((_NUM_MODES, 2), lambda i: (0, 0)),
            pl.BlockSpec((_NUM_MODES, 2), lambda i: (0, 0)),
        ],
        out_specs=pl.BlockSpec((2, _COLS), lambda i: (0, i)),
        out_shape=jax.ShapeDtypeStruct((2, _BATCH), jnp.float32),
        interpret=interpret,
    )(zt, W, embedding_table)
    return out.T


def kernel(z2_onehot, W, embedding_table):
    return _run(z2_onehot, W, embedding_table)
